# baseline (speedup 1.0000x reference)
import functools

import jax
import jax.numpy as jnp
from jax import lax
from jax.experimental import pallas as pl
from jax.experimental.pallas import tpu as pltpu
from jax.experimental.pallas import tpu_sc as plsc

_K = 1024
_D = 64
_TILE_M = 2048
_KC = 512
_COMMIT = 0.25
_CHUNKS = 2

_info = plsc.get_sparse_core_info()
_NC, _NS, _L = _info.num_cores, _info.num_subcores, _info.num_lanes
_NW = _NC * _NS
_GATHER_CHUNK = 128
_STORE_CHUNK = 512


def _vq_tc_body(n_total, z_ref, e_ref, idx_ref, loss_ref, acc_ref, d_ref):
    z = z_ref[...]
    e = e_ref[...]
    row_sq = jnp.sum(z * z, axis=1, keepdims=True)
    e_sq = jnp.sum(e * e, axis=1)
    m_run = None
    for c in range(_K // _KC):
        e_c = e[c * _KC:(c + 1) * _KC, :]
        mm_c = lax.dot_general(z, e_c, (((1,), (1,)), ((), ())),
                               preferred_element_type=jnp.float32)
        d_c = (row_sq + e_sq[c * _KC:(c + 1) * _KC]) - 2.0 * mm_c
        d_ref[:, c * _KC:(c + 1) * _KC] = d_c
        m_run = d_c if c == 0 else jnp.minimum(m_run, d_c)
    min_d = jnp.min(m_run, axis=1, keepdims=True)
    d = d_ref[...]
    k_iota = lax.broadcasted_iota(jnp.int32, d.shape, 1).astype(jnp.float32)
    idx = jnp.min(jnp.where(d == min_d, k_iota, float(_K)), axis=1,
                  keepdims=True)
    idx_ref[...] = idx.astype(jnp.int32)

    @pl.when(pl.program_id(0) == 0)
    def _():
        acc_ref[...] = jnp.zeros_like(acc_ref)
    acc_ref[...] += min_d

    @pl.when(pl.program_id(0) == pl.num_programs(0) - 1)
    def _():
        loss_ref[...] = jnp.sum(acc_ref[...], keepdims=True).reshape(1, 1) * (
            (1.0 + _COMMIT) / n_total)


def _sc_gather_body(idx_rows_per_w, table_hbm, idx_hbm, out_hbm,
                    idx_v, rows_v, sem):
    wid = lax.axis_index("s") * _NC + lax.axis_index("c")
    base_idx_row = wid * idx_rows_per_w
    base_tok = base_idx_row * 8 * _L
    pltpu.sync_copy(idx_hbm.at[pl.ds(base_tok, idx_rows_per_w * 8 * _L)], idx_v)
    n_inner = _STORE_CHUNK // _GATHER_CHUNK
    for c in range(idx_rows_per_w // n_inner):
        copies = []
        for j in range(n_inner):
            copies.append(pltpu.async_copy(
                table_hbm.at[idx_v.at[pl.ds((c * n_inner + j) * _GATHER_CHUNK,
                                            _GATHER_CHUNK)]],
                rows_v.at[pl.ds(j * _GATHER_CHUNK, _GATHER_CHUNK)],
                sem))
        for cp in copies:
            cp.wait()
        tok0 = base_tok + c * _STORE_CHUNK
        pltpu.sync_copy(
            rows_v, out_hbm.at[tok0 // 1024, pl.ds(tok0 % 1024, _STORE_CHUNK)])


def kernel(z, embedding_weight):
    B, T, D = z.shape
    M = B * T
    z_flat = z.reshape(M, D)
    Mc = M // _CHUNKS
    tiles_per_chunk = Mc // _TILE_M
    idx_rows_per_w = Mc // _NW // (8 * _L)

    mesh = plsc.VectorSubcoreMesh(core_axis_name="c", subcore_axis_name="s")
    gather = pl.kernel(
        functools.partial(_sc_gather_body, idx_rows_per_w), mesh=mesh,
        out_type=jax.ShapeDtypeStruct((Mc // 1024, 1024, _D), jnp.float32),
        scratch_types=[
            pltpu.VMEM((idx_rows_per_w * 8 * _L,), jnp.int32),
            pltpu.VMEM((_STORE_CHUNK, _D), jnp.float32),
            pltpu.SemaphoreType.DMA,
        ],
        compiler_params=pltpu.CompilerParams(use_tc_tiling_on_sc=False),
    )

    zq_parts, idx_parts, loss_parts = [], [], []
    for c in range(_CHUNKS):
        tile0 = c * tiles_per_chunk
        idx3, loss_c = pl.pallas_call(
            functools.partial(_vq_tc_body, M * _D),
            grid=(tiles_per_chunk,),
            in_specs=[
                pl.BlockSpec((_TILE_M, D), lambda i, t0=tile0: (t0 + i, 0)),
                pl.BlockSpec((_K, D), lambda i: (0, 0)),
            ],
            out_specs=[
                pl.BlockSpec((_TILE_M, 1), lambda i: (i, 0)),
                pl.BlockSpec((1, 1), lambda i: (0, 0)),
            ],
            out_shape=[
                jax.ShapeDtypeStruct((Mc, 1), jnp.int32),
                jax.ShapeDtypeStruct((1, 1), jnp.float32),
            ],
            scratch_shapes=[pltpu.VMEM((_TILE_M, 1), jnp.float32),
                            pltpu.VMEM((_TILE_M, _K), jnp.float32)],
            compiler_params=pltpu.CompilerParams(
                dimension_semantics=("arbitrary",)),
        )(z_flat, embedding_weight)

        idx_c = idx3.reshape(Mc)
        zq_parts.append(gather(embedding_weight, idx_c))
        idx_parts.append(idx_c)
        loss_parts.append(loss_c[0, 0])

    zq = zq_parts[0] if _CHUNKS == 1 else jnp.concatenate(zq_parts, axis=0)
    idx_flat = (jnp.concatenate(idx_parts) if _CHUNKS > 1 else idx_parts[0])
    loss = sum(loss_parts)
    return zq, loss, idx_flat


# speedup vs baseline: 1.0499x; 1.0499x over previous
import functools

import jax
import jax.numpy as jnp
from jax import lax
from jax.experimental import pallas as pl
from jax.experimental.pallas import tpu as pltpu
from jax.experimental.pallas import tpu_sc as plsc

_K = 1024
_D = 64
_TILE_M = 2048
_KC = 512
_COMMIT = 0.25
_CHUNKS = 1

_info = plsc.get_sparse_core_info()
_NC, _NS, _L = _info.num_cores, _info.num_subcores, _info.num_lanes
_NW = _NC * _NS
_GATHER_CHUNK = 128
_STORE_CHUNK = 512


def _vq_tc_body(n_total, z_ref, e_ref, idx_ref, loss_ref, acc_ref, d_ref):
    z = z_ref[...]
    e = e_ref[...]
    row_sq = jnp.sum(z * z, axis=1, keepdims=True)
    e_sq = jnp.sum(e * e, axis=1)
    m_run = None
    for c in range(_K // _KC):
        e_c = e[c * _KC:(c + 1) * _KC, :]
        mm_c = lax.dot_general(z, e_c, (((1,), (1,)), ((), ())),
                               preferred_element_type=jnp.float32)
        d_c = (row_sq + e_sq[c * _KC:(c + 1) * _KC]) - 2.0 * mm_c
        d_ref[:, c * _KC:(c + 1) * _KC] = d_c
        m_run = d_c if c == 0 else jnp.minimum(m_run, d_c)
    min_d = jnp.min(m_run, axis=1, keepdims=True)
    d = d_ref[...]
    k_iota = lax.broadcasted_iota(jnp.int32, d.shape, 1).astype(jnp.float32)
    idx = jnp.min(jnp.where(d == min_d, k_iota, float(_K)), axis=1,
                  keepdims=True)
    idx_ref[...] = idx.astype(jnp.int32)

    @pl.when(pl.program_id(0) == 0)
    def _():
        acc_ref[...] = jnp.zeros_like(acc_ref)
    acc_ref[...] += min_d

    @pl.when(pl.program_id(0) == pl.num_programs(0) - 1)
    def _():
        loss_ref[...] = jnp.sum(acc_ref[...], keepdims=True).reshape(1, 1) * (
            (1.0 + _COMMIT) / n_total)


def _sc_gather_body(idx_rows_per_w, table_hbm, idx_hbm, out_hbm,
                    idx_v, rows_v, sem):
    wid = lax.axis_index("s") * _NC + lax.axis_index("c")
    base_idx_row = wid * idx_rows_per_w
    base_tok = base_idx_row * 8 * _L
    pltpu.sync_copy(idx_hbm.at[pl.ds(base_tok, idx_rows_per_w * 8 * _L)], idx_v)
    n_inner = _STORE_CHUNK // _GATHER_CHUNK
    for c in range(idx_rows_per_w // n_inner):
        copies = []
        for j in range(n_inner):
            copies.append(pltpu.async_copy(
                table_hbm.at[idx_v.at[pl.ds((c * n_inner + j) * _GATHER_CHUNK,
                                            _GATHER_CHUNK)]],
                rows_v.at[pl.ds(j * _GATHER_CHUNK, _GATHER_CHUNK)],
                sem))
        for cp in copies:
            cp.wait()
        tok0 = base_tok + c * _STORE_CHUNK
        pltpu.sync_copy(
            rows_v, out_hbm.at[tok0 // 1024, pl.ds(tok0 % 1024, _STORE_CHUNK)])


def kernel(z, embedding_weight):
    B, T, D = z.shape
    M = B * T
    z_flat = z.reshape(M, D)
    Mc = M // _CHUNKS
    tiles_per_chunk = Mc // _TILE_M
    idx_rows_per_w = Mc // _NW // (8 * _L)

    mesh = plsc.VectorSubcoreMesh(core_axis_name="c", subcore_axis_name="s")
    gather = pl.kernel(
        functools.partial(_sc_gather_body, idx_rows_per_w), mesh=mesh,
        out_type=jax.ShapeDtypeStruct((Mc // 1024, 1024, _D), jnp.float32),
        scratch_types=[
            pltpu.VMEM((idx_rows_per_w * 8 * _L,), jnp.int32),
            pltpu.VMEM((_STORE_CHUNK, _D), jnp.float32),
            pltpu.SemaphoreType.DMA,
        ],
        compiler_params=pltpu.CompilerParams(use_tc_tiling_on_sc=False),
    )

    zq_parts, idx_parts, loss_parts = [], [], []
    for c in range(_CHUNKS):
        tile0 = c * tiles_per_chunk
        idx3, loss_c = pl.pallas_call(
            functools.partial(_vq_tc_body, M * _D),
            grid=(tiles_per_chunk,),
            in_specs=[
                pl.BlockSpec((_TILE_M, D), lambda i, t0=tile0: (t0 + i, 0)),
                pl.BlockSpec((_K, D), lambda i: (0, 0)),
            ],
            out_specs=[
                pl.BlockSpec((_TILE_M, 1), lambda i: (i, 0)),
                pl.BlockSpec((1, 1), lambda i: (0, 0)),
            ],
            out_shape=[
                jax.ShapeDtypeStruct((Mc, 1), jnp.int32),
                jax.ShapeDtypeStruct((1, 1), jnp.float32),
            ],
            scratch_shapes=[pltpu.VMEM((_TILE_M, 1), jnp.float32),
                            pltpu.VMEM((_TILE_M, _K), jnp.float32)],
            compiler_params=pltpu.CompilerParams(
                dimension_semantics=("arbitrary",)),
        )(z_flat, embedding_weight)

        idx_c = idx3.reshape(Mc)
        zq_parts.append(gather(embedding_weight, idx_c))
        idx_parts.append(idx_c)
        loss_parts.append(loss_c[0, 0])

    zq = zq_parts[0] if _CHUNKS == 1 else jnp.concatenate(zq_parts, axis=0)
    idx_flat = (jnp.concatenate(idx_parts) if _CHUNKS > 1 else idx_parts[0])
    loss = sum(loss_parts)
    return zq, loss, idx_flat


# speedup vs baseline: 1.0614x; 1.0110x over previous
import functools

import jax
import jax.numpy as jnp
from jax import lax
from jax.experimental import pallas as pl
from jax.experimental.pallas import tpu as pltpu
from jax.experimental.pallas import tpu_sc as plsc

_K = 1024
_D = 64
_TILE_M = 4096
_KC = 512
_COMMIT = 0.25
_CHUNKS = 1

_info = plsc.get_sparse_core_info()
_NC, _NS, _L = _info.num_cores, _info.num_subcores, _info.num_lanes
_NW = _NC * _NS
_GATHER_CHUNK = 128
_STORE_CHUNK = 512


def _vq_tc_body(n_total, z_ref, e_ref, idx_ref, loss_ref, acc_ref, d_ref):
    z = z_ref[...]
    e = e_ref[...]
    row_sq = jnp.sum(z * z, axis=1, keepdims=True)
    e_sq = jnp.sum(e * e, axis=1)
    m_run = None
    for c in range(_K // _KC):
        e_c = e[c * _KC:(c + 1) * _KC, :]
        mm_c = lax.dot_general(z, e_c, (((1,), (1,)), ((), ())),
                               preferred_element_type=jnp.float32)
        d_c = (row_sq + e_sq[c * _KC:(c + 1) * _KC]) - 2.0 * mm_c
        d_ref[:, c * _KC:(c + 1) * _KC] = d_c
        m_run = d_c if c == 0 else jnp.minimum(m_run, d_c)
    min_d = jnp.min(m_run, axis=1, keepdims=True)
    d = d_ref[...]
    k_iota = lax.broadcasted_iota(jnp.int32, d.shape, 1).astype(jnp.float32)
    idx = jnp.min(jnp.where(d == min_d, k_iota, float(_K)), axis=1,
                  keepdims=True)
    idx_ref[...] = idx.astype(jnp.int32)

    @pl.when(pl.program_id(0) == 0)
    def _():
        acc_ref[...] = jnp.zeros_like(acc_ref)
    acc_ref[...] += min_d

    @pl.when(pl.program_id(0) == pl.num_programs(0) - 1)
    def _():
        loss_ref[...] = jnp.sum(acc_ref[...], keepdims=True).reshape(1, 1) * (
            (1.0 + _COMMIT) / n_total)


def _sc_gather_body(idx_rows_per_w, table_hbm, idx_hbm, out_hbm,
                    idx_v, rows_v, sem):
    wid = lax.axis_index("s") * _NC + lax.axis_index("c")
    base_idx_row = wid * idx_rows_per_w
    base_tok = base_idx_row * 8 * _L
    pltpu.sync_copy(idx_hbm.at[pl.ds(base_tok, idx_rows_per_w * 8 * _L)], idx_v)
    n_inner = _STORE_CHUNK // _GATHER_CHUNK
    for c in range(idx_rows_per_w // n_inner):
        copies = []
        for j in range(n_inner):
            copies.append(pltpu.async_copy(
                table_hbm.at[idx_v.at[pl.ds((c * n_inner + j) * _GATHER_CHUNK,
                                            _GATHER_CHUNK)]],
                rows_v.at[pl.ds(j * _GATHER_CHUNK, _GATHER_CHUNK)],
                sem))
        for cp in copies:
            cp.wait()
        tok0 = base_tok + c * _STORE_CHUNK
        pltpu.sync_copy(
            rows_v, out_hbm.at[tok0 // 1024, pl.ds(tok0 % 1024, _STORE_CHUNK)])


def kernel(z, embedding_weight):
    B, T, D = z.shape
    M = B * T
    z_flat = z.reshape(M, D)
    Mc = M // _CHUNKS
    tiles_per_chunk = Mc // _TILE_M
    idx_rows_per_w = Mc // _NW // (8 * _L)

    mesh = plsc.VectorSubcoreMesh(core_axis_name="c", subcore_axis_name="s")
    gather = pl.kernel(
        functools.partial(_sc_gather_body, idx_rows_per_w), mesh=mesh,
        out_type=jax.ShapeDtypeStruct((Mc // 1024, 1024, _D), jnp.float32),
        scratch_types=[
            pltpu.VMEM((idx_rows_per_w * 8 * _L,), jnp.int32),
            pltpu.VMEM((_STORE_CHUNK, _D), jnp.float32),
            pltpu.SemaphoreType.DMA,
        ],
        compiler_params=pltpu.CompilerParams(use_tc_tiling_on_sc=False),
    )

    zq_parts, idx_parts, loss_parts = [], [], []
    for c in range(_CHUNKS):
        tile0 = c * tiles_per_chunk
        idx3, loss_c = pl.pallas_call(
            functools.partial(_vq_tc_body, M * _D),
            grid=(tiles_per_chunk,),
            in_specs=[
                pl.BlockSpec((_TILE_M, D), lambda i, t0=tile0: (t0 + i, 0)),
                pl.BlockSpec((_K, D), lambda i: (0, 0)),
            ],
            out_specs=[
                pl.BlockSpec((_TILE_M, 1), lambda i: (i, 0)),
                pl.BlockSpec((1, 1), lambda i: (0, 0)),
            ],
            out_shape=[
                jax.ShapeDtypeStruct((Mc, 1), jnp.int32),
                jax.ShapeDtypeStruct((1, 1), jnp.float32),
            ],
            scratch_shapes=[pltpu.VMEM((_TILE_M, 1), jnp.float32),
                            pltpu.VMEM((_TILE_M, _K), jnp.float32)],
            compiler_params=pltpu.CompilerParams(
                dimension_semantics=("arbitrary",)),
        )(z_flat, embedding_weight)

        idx_c = idx3.reshape(Mc)
        zq_parts.append(gather(embedding_weight, idx_c))
        idx_parts.append(idx_c)
        loss_parts.append(loss_c[0, 0])

    zq = zq_parts[0] if _CHUNKS == 1 else jnp.concatenate(zq_parts, axis=0)
    idx_flat = (jnp.concatenate(idx_parts) if _CHUNKS > 1 else idx_parts[0])
    loss = sum(loss_parts)
    return zq, loss, idx_flat
